# baseline (speedup 1.0000x reference)
import functools
import jax
import jax.numpy as jnp
from jax import lax
from jax.experimental import pallas as pl
from jax.experimental.pallas import tpu as pltpu
from jax.experimental.pallas import tpu_sc as plsc

NUM_EXPERTS = 64
TOP_K = 8
D_MODEL = 4096
N_TOKENS = 16384
BN = 1024

NC, NS, L = 2, 16, 16
NW = NC * NS
TPW = N_TOKENS // NW
NG = TPW // L

_SORT8 = ((0, 1), (2, 3), (4, 5), (6, 7),
          (0, 2), (1, 3), (4, 6), (5, 7),
          (1, 2), (5, 6),
          (0, 4), (1, 5), (2, 6), (3, 7),
          (2, 4), (3, 5),
          (1, 2), (3, 4), (5, 6))
_BIT8 = ((0, 4), (1, 5), (2, 6), (3, 7),
         (0, 2), (1, 3), (4, 6), (5, 7),
         (0, 1), (2, 3), (4, 5), (6, 7))


def _logits_body(x_ref, w_ref, b_ref, lt_ref):
    lt_ref[...] = jax.lax.dot_general(
        w_ref[...], x_ref[...],
        dimension_numbers=(((1,), (1,)), ((), ())),
        preferred_element_type=jnp.float32,
    ) + b_ref[...]


def _tc_logits(x, W, b2d, n):
    return pl.pallas_call(
        _logits_body,
        grid=(n // BN,),
        in_specs=[
            pl.BlockSpec((BN, D_MODEL), lambda i: (i, 0)),
            pl.BlockSpec((NUM_EXPERTS, D_MODEL), lambda i: (0, 0)),
            pl.BlockSpec((NUM_EXPERTS, 1), lambda i: (0, 0)),
        ],
        out_specs=pl.BlockSpec((NUM_EXPERTS, BN), lambda i: (0, i)),
        out_shape=jax.ShapeDtypeStruct((NUM_EXPERTS, n), jnp.float32),
    )(x, W, b2d)


def _sc_key(v, e):
    i = lax.bitcast_convert_type(v, jnp.int32)
    k = i ^ (lax.shift_right_arithmetic(i, 31) & jnp.int32(0x7FFFFFFF))
    return (k & jnp.int32(-64)) | jnp.int32(63 - e)


def _sc_unkey(k):
    i = k ^ (lax.shift_right_arithmetic(k, 31) & jnp.int32(0x7FFFFFFF))
    return lax.bitcast_convert_type(i, jnp.float32)


def _route_body(lt_hbm, out_hbm, acc_hbm, lt_v, wbuf, out_v, acc_v,
                sem):
    wid = lax.axis_index("s") * NC + lax.axis_index("c")
    base = wid * TPW
    pltpu.sync_copy(lt_hbm.at[:, pl.ds(base, TPW)], lt_v)

    iota = lax.iota(jnp.int32, L)
    zero = jnp.zeros((L,), jnp.float32)
    for e in range(NUM_EXPERTS):
        acc_v[e, :] = zero

    def group(g, _):
        goff = g * L
        t = None
        for c in range(NUM_EXPERTS // 8):
            k = [_sc_key(lt_v[c * 8 + e8, pl.ds(goff, L)], c * 8 + e8)
                 for e8 in range(8)]
            for i, j in _SORT8:
                hi = jnp.maximum(k[i], k[j])
                lo = jnp.minimum(k[i], k[j])
                k[i], k[j] = hi, lo
            if t is None:
                t = k
            else:
                t = [jnp.maximum(t[i], k[7 - i]) for i in range(8)]
                for i, j in _BIT8:
                    hi = jnp.maximum(t[i], t[j])
                    lo = jnp.minimum(t[i], t[j])
                    t[i], t[j] = hi, lo
        thr = t[TOP_K - 1]
        m0 = _sc_unkey(t[0])
        dn = [zero, zero, zero, zero]
        for e in range(NUM_EXPERTS):
            v = lt_v[e, pl.ds(goff, L)]
            w = jnp.where(_sc_key(v, e) >= thr, jnp.exp(v - m0), 0.0)
            wbuf[e, :] = w
            dn[e % 4] = dn[e % 4] + w
        rden = 1.0 / ((dn[0] + dn[1]) + (dn[2] + dn[3]))
        for e in range(NUM_EXPERTS):
            w = wbuf[e, :] * rden
            acc_v[e, :] = acc_v[e, :] + w
            out_v[e, pl.ds(goff, L)] = w
        return ()

    lax.fori_loop(0, NG, group, (), unroll=2)
    pltpu.sync_copy(out_v, out_hbm.at[:, pl.ds(base, TPW)])
    pltpu.sync_copy(acc_v, acc_hbm.at[wid])


def _sc_route(lt, n):
    mesh = plsc.VectorSubcoreMesh(core_axis_name="c", subcore_axis_name="s",
                                  num_cores=NC, num_subcores=NS)
    f = pl.kernel(
        _route_body,
        out_type=[
            jax.ShapeDtypeStruct((NUM_EXPERTS, n), jnp.float32),
            jax.ShapeDtypeStruct((NW, NUM_EXPERTS, L), jnp.float32),
        ],
        mesh=mesh,
        scratch_types=[
            pltpu.VMEM((NUM_EXPERTS, TPW), jnp.float32),
            pltpu.VMEM((NUM_EXPERTS, L), jnp.float32),
            pltpu.VMEM((NUM_EXPERTS, TPW), jnp.float32),
            pltpu.VMEM((NUM_EXPERTS, L), jnp.float32),
            pltpu.SemaphoreType.DMA,
        ],
    )
    return f(lt)


def _final_body(rt_ref, acc_ref, out_ref, loss_ref):
    i = pl.program_id(0)
    out_ref[...] = rt_ref[...].T

    @pl.when(i == pl.num_programs(0) - 1)
    def _():
        cs = jnp.sum(acc_ref[...], axis=(0, 2), keepdims=True)
        total = jnp.sum(cs)
        usage = cs / total
        loss_ref[...] = jnp.sum((usage - 1.0 / NUM_EXPERTS) ** 2,
                                keepdims=True).reshape(1, 1)


def _tc_finalize(rt, acc, n):
    return pl.pallas_call(
        _final_body,
        grid=(n // BN,),
        in_specs=[
            pl.BlockSpec((NUM_EXPERTS, BN), lambda i: (0, i)),
            pl.BlockSpec((NW, NUM_EXPERTS, L), lambda i: (0, 0, 0)),
        ],
        out_specs=[
            pl.BlockSpec((BN, NUM_EXPERTS), lambda i: (i, 0)),
            pl.BlockSpec((1, 1), lambda i: (0, 0)),
        ],
        out_shape=[
            jax.ShapeDtypeStruct((n, NUM_EXPERTS), jnp.float32),
            jax.ShapeDtypeStruct((1, 1), jnp.float32),
        ],
    )(rt, acc)


def kernel(x, W, b):
    n = x.shape[0]
    lt = _tc_logits(x, W, b.reshape(NUM_EXPERTS, 1), n)
    rt, acc = _sc_route(lt, n)
    routing, loss = _tc_finalize(rt, acc, n)
    return routing, loss[0, 0]


# speedup vs baseline: 1.0497x; 1.0497x over previous
import functools
import jax
import jax.numpy as jnp
from jax import lax
from jax.experimental import pallas as pl
from jax.experimental.pallas import tpu as pltpu
from jax.experimental.pallas import tpu_sc as plsc

NUM_EXPERTS = 64
TOP_K = 8
D_MODEL = 4096
N_TOKENS = 16384
BN = 1024

NC, NS, L = 2, 16, 16
NW = NC * NS
TPW = N_TOKENS // NW
NG = TPW // L

_SORT8 = ((0, 1), (2, 3), (4, 5), (6, 7),
          (0, 2), (1, 3), (4, 6), (5, 7),
          (1, 2), (5, 6),
          (0, 4), (1, 5), (2, 6), (3, 7),
          (2, 4), (3, 5),
          (1, 2), (3, 4), (5, 6))
_BIT8 = ((0, 4), (1, 5), (2, 6), (3, 7),
         (0, 2), (1, 3), (4, 6), (5, 7),
         (0, 1), (2, 3), (4, 5), (6, 7))


def _logits_body(x_ref, w_ref, b_ref, lt_ref):
    lt_ref[...] = jax.lax.dot_general(
        w_ref[...], x_ref[...],
        dimension_numbers=(((1,), (1,)), ((), ())),
        preferred_element_type=jnp.float32,
    ) + b_ref[...]


def _tc_logits(x, W, b2d, n):
    return pl.pallas_call(
        _logits_body,
        grid=(n // BN,),
        in_specs=[
            pl.BlockSpec((BN, D_MODEL), lambda i: (i, 0)),
            pl.BlockSpec((NUM_EXPERTS, D_MODEL), lambda i: (0, 0)),
            pl.BlockSpec((NUM_EXPERTS, 1), lambda i: (0, 0)),
        ],
        out_specs=pl.BlockSpec((NUM_EXPERTS, BN), lambda i: (0, i)),
        out_shape=jax.ShapeDtypeStruct((NUM_EXPERTS, n), jnp.float32),
    )(x, W, b2d)


def _sc_key(v, e):
    i = lax.bitcast_convert_type(v, jnp.int32)
    k = i ^ (lax.shift_right_arithmetic(i, 31) & jnp.int32(0x7FFFFFFF))
    return (k & jnp.int32(-64)) | jnp.int32(63 - e)


def _sc_unkey(k):
    i = k ^ (lax.shift_right_arithmetic(k, 31) & jnp.int32(0x7FFFFFFF))
    return lax.bitcast_convert_type(i, jnp.float32)


def _route_body(lt_hbm, out_hbm, acc_hbm, lt_v, wbuf, out_v, acc_v,
                sem):
    wid = lax.axis_index("s") * NC + lax.axis_index("c")
    base = wid * TPW
    pltpu.sync_copy(lt_hbm.at[:, pl.ds(base, TPW)], lt_v)

    iota = lax.iota(jnp.int32, L)
    zero = jnp.zeros((L,), jnp.float32)
    for e in range(NUM_EXPERTS):
        acc_v[e, :] = zero

    def group(g, _):
        goff = g * L
        t = None
        for c in range(NUM_EXPERTS // 8):
            k = [_sc_key(lt_v[c * 8 + e8, pl.ds(goff, L)], c * 8 + e8)
                 for e8 in range(8)]
            for i, j in _SORT8:
                hi = jnp.maximum(k[i], k[j])
                lo = jnp.minimum(k[i], k[j])
                k[i], k[j] = hi, lo
            if t is None:
                t = k
            else:
                t = [jnp.maximum(t[i], k[7 - i]) for i in range(8)]
                for i, j in _BIT8:
                    hi = jnp.maximum(t[i], t[j])
                    lo = jnp.minimum(t[i], t[j])
                    t[i], t[j] = hi, lo
        thr = t[TOP_K - 1]
        m0 = _sc_unkey(t[0])
        dn = [zero, zero, zero, zero]
        for e in range(NUM_EXPERTS):
            v = lt_v[e, pl.ds(goff, L)]
            w = jnp.where(_sc_key(v, e) >= thr, jnp.exp(v - m0), 0.0)
            wbuf[e, :] = w
            dn[e % 4] = dn[e % 4] + w
        rden = 1.0 / ((dn[0] + dn[1]) + (dn[2] + dn[3]))
        for e in range(NUM_EXPERTS):
            w = wbuf[e, :] * rden
            acc_v[e, :] = acc_v[e, :] + w
            out_v[e, pl.ds(goff, L)] = w
        return ()

    lax.fori_loop(0, NG, group, (), unroll=False)
    pltpu.sync_copy(out_v, out_hbm.at[:, pl.ds(base, TPW)])
    pltpu.sync_copy(acc_v, acc_hbm.at[wid])


def _sc_route(lt, n):
    mesh = plsc.VectorSubcoreMesh(core_axis_name="c", subcore_axis_name="s",
                                  num_cores=NC, num_subcores=NS)
    f = pl.kernel(
        _route_body,
        out_type=[
            jax.ShapeDtypeStruct((NUM_EXPERTS, n), jnp.float32),
            jax.ShapeDtypeStruct((NW, NUM_EXPERTS, L), jnp.float32),
        ],
        mesh=mesh,
        scratch_types=[
            pltpu.VMEM((NUM_EXPERTS, TPW), jnp.float32),
            pltpu.VMEM((NUM_EXPERTS, L), jnp.float32),
            pltpu.VMEM((NUM_EXPERTS, TPW), jnp.float32),
            pltpu.VMEM((NUM_EXPERTS, L), jnp.float32),
            pltpu.SemaphoreType.DMA,
        ],
    )
    return f(lt)


def _final_body(rt_ref, acc_ref, out_ref, loss_ref):
    i = pl.program_id(0)
    out_ref[...] = rt_ref[...].T

    @pl.when(i == pl.num_programs(0) - 1)
    def _():
        cs = jnp.sum(acc_ref[...], axis=(0, 2), keepdims=True)
        total = jnp.sum(cs)
        usage = cs / total
        loss_ref[...] = jnp.sum((usage - 1.0 / NUM_EXPERTS) ** 2,
                                keepdims=True).reshape(1, 1)


def _tc_finalize(rt, acc, n):
    return pl.pallas_call(
        _final_body,
        grid=(n // BN,),
        in_specs=[
            pl.BlockSpec((NUM_EXPERTS, BN), lambda i: (0, i)),
            pl.BlockSpec((NW, NUM_EXPERTS, L), lambda i: (0, 0, 0)),
        ],
        out_specs=[
            pl.BlockSpec((BN, NUM_EXPERTS), lambda i: (i, 0)),
            pl.BlockSpec((1, 1), lambda i: (0, 0)),
        ],
        out_shape=[
            jax.ShapeDtypeStruct((n, NUM_EXPERTS), jnp.float32),
            jax.ShapeDtypeStruct((1, 1), jnp.float32),
        ],
    )(rt, acc)


def kernel(x, W, b):
    n = x.shape[0]
    lt = _tc_logits(x, W, b.reshape(NUM_EXPERTS, 1), n)
    rt, acc = _sc_route(lt, n)
    routing, loss = _tc_finalize(rt, acc, n)
    return routing, loss[0, 0]


# speedup vs baseline: 1.5980x; 1.5223x over previous
import functools
import jax
import jax.numpy as jnp
from jax import lax
from jax.experimental import pallas as pl
from jax.experimental.pallas import tpu as pltpu
from jax.experimental.pallas import tpu_sc as plsc

NUM_EXPERTS = 64
TOP_K = 8
D_MODEL = 4096
N_TOKENS = 16384
BN = 1024

NC, NS, L = 2, 16, 16
NW = NC * NS
TPW = N_TOKENS // NW
NG = TPW // L

_SORT8 = ((0, 1), (2, 3), (4, 5), (6, 7),
          (0, 2), (1, 3), (4, 6), (5, 7),
          (1, 2), (5, 6),
          (0, 4), (1, 5), (2, 6), (3, 7),
          (2, 4), (3, 5),
          (1, 2), (3, 4), (5, 6))
_BIT8 = ((0, 4), (1, 5), (2, 6), (3, 7),
         (0, 2), (1, 3), (4, 6), (5, 7),
         (0, 1), (2, 3), (4, 5), (6, 7))


def _logits_body(x_ref, w_ref, b_ref, lt_ref):
    lt_ref[...] = jax.lax.dot_general(
        w_ref[...], x_ref[...],
        dimension_numbers=(((1,), (1,)), ((), ())),
        preferred_element_type=jnp.float32,
    ) + b_ref[...]


def _tc_logits(x, W, b2d, n):
    return pl.pallas_call(
        _logits_body,
        grid=(n // BN,),
        in_specs=[
            pl.BlockSpec((BN, D_MODEL), lambda i: (i, 0)),
            pl.BlockSpec((NUM_EXPERTS, D_MODEL), lambda i: (0, 0)),
            pl.BlockSpec((NUM_EXPERTS, 1), lambda i: (0, 0)),
        ],
        out_specs=pl.BlockSpec((NUM_EXPERTS, BN), lambda i: (0, i)),
        out_shape=jax.ShapeDtypeStruct((NUM_EXPERTS, n), jnp.float32),
    )(x, W, b2d)


def _sc_key(v, e):
    i = lax.bitcast_convert_type(v, jnp.int32)
    return lax.bitcast_convert_type((i & jnp.int32(-64)) | jnp.int32(63 - e),
                                    jnp.float32)


def _route_body(lt_hbm, out_hbm, acc_hbm, lt_v, out_v, acc_v, sem):
    wid = lax.axis_index("s") * NC + lax.axis_index("c")
    base = wid * TPW
    pltpu.sync_copy(lt_hbm.at[:, pl.ds(base, TPW)], lt_v)

    iota = lax.iota(jnp.int32, L)
    zero = jnp.zeros((L,), jnp.float32)

    def zbody(i, _):
        for u in range(16):
            out_v[pl.ds(i * 256 + u * L, L)] = zero
        return ()

    lax.fori_loop(0, (TPW * NUM_EXPERTS) // 256, zbody, (), unroll=False)
    for e in range(NUM_EXPERTS):
        acc_v[pl.ds(e * L, L)] = zero

    def group(g, _):
        goff = g * L
        t = None
        for c in range(NUM_EXPERTS // 8):
            k = [_sc_key(lt_v[c * 8 + e8, pl.ds(goff, L)], c * 8 + e8)
                 for e8 in range(8)]
            for i, j in _SORT8:
                hi = jnp.maximum(k[i], k[j])
                lo = jnp.minimum(k[i], k[j])
                k[i], k[j] = hi, lo
            if t is None:
                t = k
            else:
                t = [jnp.maximum(t[i], k[7 - i]) for i in range(8)]
                if c < NUM_EXPERTS // 8 - 1:
                    for i, j in _BIT8:
                        hi = jnp.maximum(t[i], t[j])
                        lo = jnp.minimum(t[i], t[j])
                        t[i], t[j] = hi, lo
        m01 = jnp.maximum(t[0], t[1])
        m23 = jnp.maximum(t[2], t[3])
        m45 = jnp.maximum(t[4], t[5])
        m67 = jnp.maximum(t[6], t[7])
        m0 = jnp.maximum(jnp.maximum(m01, m23), jnp.maximum(m45, m67))
        w = [jnp.exp(t[j] - m0) for j in range(TOP_K)]
        dn = ((w[0] + w[1]) + (w[2] + w[3])) + ((w[4] + w[5]) + (w[6] + w[7]))
        rden = 1.0 / dn
        bidx = iota * NUM_EXPERTS + (goff * NUM_EXPERTS)
        for j in range(TOP_K):
            kb = lax.bitcast_convert_type(t[j], jnp.int32)
            ej = (kb & jnp.int32(63)) ^ jnp.int32(63)
            wn = w[j] * rden
            plsc.store_scatter(out_v, [bidx + ej], wn)
            plsc.addupdate_scatter(acc_v, [ej * jnp.int32(L) + iota], wn)
        return ()

    lax.fori_loop(0, NG, group, (), unroll=False)
    pltpu.sync_copy(out_v,
                    out_hbm.at[pl.ds(base * NUM_EXPERTS, TPW * NUM_EXPERTS)])
    pltpu.sync_copy(acc_v, acc_hbm.at[wid])


def _sc_route(lt, n):
    mesh = plsc.VectorSubcoreMesh(core_axis_name="c", subcore_axis_name="s",
                                  num_cores=NC, num_subcores=NS)
    f = pl.kernel(
        _route_body,
        out_type=[
            jax.ShapeDtypeStruct((n * NUM_EXPERTS,), jnp.float32),
            jax.ShapeDtypeStruct((NW, NUM_EXPERTS * L), jnp.float32),
        ],
        mesh=mesh,
        compiler_params=pltpu.CompilerParams(needs_layout_passes=False),
        scratch_types=[
            pltpu.VMEM((NUM_EXPERTS, TPW), jnp.float32),
            pltpu.VMEM((TPW * NUM_EXPERTS,), jnp.float32),
            pltpu.VMEM((NUM_EXPERTS * L,), jnp.float32),
            pltpu.SemaphoreType.DMA,
        ],
    )
    return f(lt)


def _loss_body(acc_ref, loss_ref):
    cs = jnp.sum(acc_ref[...], axis=(0, 2), keepdims=True)
    total = jnp.sum(cs)
    usage = cs / total
    loss_ref[...] = jnp.sum((usage - 1.0 / NUM_EXPERTS) ** 2,
                            keepdims=True).reshape(1, 1)


def _tc_loss(acc):
    return pl.pallas_call(
        _loss_body,
        out_shape=jax.ShapeDtypeStruct((1, 1), jnp.float32),
    )(acc)


def kernel(x, W, b):
    n = x.shape[0]
    lt = _tc_logits(x, W, b.reshape(NUM_EXPERTS, 1), n)
    rt_flat, acc = _sc_route(lt, n)
    routing = rt_flat.reshape(n, NUM_EXPERTS)
    loss = _tc_loss(acc.reshape(NW, NUM_EXPERTS, L))
    return routing, loss[0, 0]
